# baseline (speedup 1.0000x reference)
import functools

import jax
import jax.numpy as jnp
from jax import lax
from jax.experimental import pallas as pl
from jax.experimental.pallas import tpu as pltpu
from jax.experimental.pallas import tpu_sc as plsc

B = 4096
S = 200
D = 64
V = 1000000

NC, NS = 2, 16
NW = NC * NS
BW = B // NW
L = 16
CP = BW + 1

_mesh = plsc.VectorSubcoreMesh(
    core_axis_name="c", subcore_axis_name="s", num_cores=NC, num_subcores=NS
)


@functools.partial(
    pl.kernel,
    out_type=jax.ShapeDtypeStruct((S, D // 8, NW, 8, BW), jnp.float32),
    mesh=_mesh,
    compiler_params=pltpu.CompilerParams(
        use_tc_tiling_on_sc=False, needs_layout_passes=False
    ),
    scratch_types=[
        pltpu.VMEM((S, BW), jnp.int32),
        pltpu.VMEM((2, BW), jnp.int32),
        pltpu.VMEM((2, BW, 80), jnp.float32),
        pltpu.VMEM((2, D // 8, 8, CP), jnp.float32),
        pltpu.VMEM((2, D), jnp.float32),
        pltpu.SemaphoreType.DMA,
        pltpu.SemaphoreType.DMA,
        pltpu.SemaphoreType.DMA,
        pltpu.SemaphoreType.DMA,
    ],
)
def _emb_kernel(idx_hbm, tt_hbm, pos_hbm, out_hbm,
                idx_v, jidx_v, big_v, oslab_v, posrow_v,
                gsem, psem, osem0, osem1):
    wid = lax.axis_index("s") * NC + lax.axis_index("c")
    b0 = wid * BW

    pltpu.sync_copy(idx_hbm.at[:, pl.ds(b0, BW)], idx_v)

    iota = lax.iota(jnp.int32, L)
    rv = jnp.bitwise_and(iota, 7)
    dtv = [lax.shift_right_logical(iota, 3) + 2 * k for k in range(D // L)]

    def prep_and_fire(s, buf):
        for c in range(BW // L):
            jidx_v[buf, pl.ds(c * L, L)] = idx_v[s, pl.ds(c * L, L)]
        pltpu.async_copy(tt_hbm.at[jidx_v.at[buf]], big_v.at[buf], gsem)
        pltpu.async_copy(pos_hbm.at[s], posrow_v.at[buf], psem)

    prep_and_fire(0, 0)

    def pair_body(s2, carry):
        for buf in range(2):
            s = s2 * 2 + buf
            osem = osem0 if buf == 0 else osem1

            @pl.when(s < S - 1)
            def _():
                prep_and_fire(s + 1, 1 - buf)

            pltpu.make_async_copy(
                tt_hbm.at[jidx_v.at[buf]], big_v.at[buf], gsem
            ).wait()
            pltpu.make_async_copy(pos_hbm.at[s], posrow_v.at[buf], psem).wait()

            @pl.when(s >= 2)
            def _():
                pltpu.make_async_copy(
                    oslab_v.at[buf, :, :, pl.ds(0, BW)],
                    out_hbm.at[s - 2, :, wid],
                    osem,
                ).wait()

            pos_k = [posrow_v[buf, pl.ds(k * L, L)] for k in range(D // L)]

            def c_body(c):
                for j in range(L):
                    t = c * L + j
                    tv = lax.broadcast(t, (L,))
                    for k in range(D // L):
                        v = big_v[buf, t, pl.ds(k * L, L)] + pos_k[k]
                        plsc.store_scatter(
                            oslab_v.at[buf], [dtv[k], rv, tv], v
                        )

            plsc.parallel_loop(0, BW // L, 1, unroll=2)(c_body)

            pltpu.async_copy(
                oslab_v.at[buf, :, :, pl.ds(0, BW)], out_hbm.at[s, :, wid], osem
            )
        return carry

    lax.fori_loop(0, S // 2, pair_body, 0)

    pltpu.make_async_copy(
        oslab_v.at[0, :, :, pl.ds(0, BW)], out_hbm.at[S - 2, :, wid], osem0
    ).wait()
    pltpu.make_async_copy(
        oslab_v.at[1, :, :, pl.ds(0, BW)], out_hbm.at[S - 1, :, wid], osem1
    ).wait()


def kernel(inputs, token_table, pos_table):
    idx_t = inputs.T.astype(jnp.int32)
    tt = jnp.concatenate([token_table, token_table[:, :16]], axis=1)
    out5 = _emb_kernel(idx_t, tt, pos_table)
    return out5.transpose(2, 4, 0, 1, 3).reshape(B, S, D)


# speedup vs baseline: 1.5868x; 1.5868x over previous
import functools

import jax
import jax.numpy as jnp
from jax import lax
from jax.experimental import pallas as pl
from jax.experimental.pallas import tpu as pltpu
from jax.experimental.pallas import tpu_sc as plsc

B = 4096
S = 200
D = 64
V = 1000000

NC, NS = 2, 16
NW = NC * NS
BW = B // NW
L = 16
CP = BW + 1

_mesh = plsc.VectorSubcoreMesh(
    core_axis_name="c", subcore_axis_name="s", num_cores=NC, num_subcores=NS
)


@functools.partial(
    pl.kernel,
    out_type=jax.ShapeDtypeStruct((S, D // 8, NW, 8, BW), jnp.float32),
    mesh=_mesh,
    compiler_params=pltpu.CompilerParams(
        use_tc_tiling_on_sc=False, needs_layout_passes=False
    ),
    scratch_types=[
        pltpu.VMEM((S, BW), jnp.int32),
        pltpu.VMEM((2, BW), jnp.int32),
        pltpu.VMEM((2, BW, 2 * D), jnp.float32),
        pltpu.VMEM((2, D // 8, 8, CP), jnp.float32),
        pltpu.VMEM((2, D), jnp.float32),
        pltpu.SemaphoreType.DMA,
        pltpu.SemaphoreType.DMA,
        pltpu.SemaphoreType.DMA,
        pltpu.SemaphoreType.DMA,
    ],
)
def _emb_kernel(idx_hbm, tt_hbm, pos_hbm, out_hbm,
                idx_v, jidx_v, big_v, oslab_v, posrow_v,
                gsem, psem, osem0, osem1):
    wid = lax.axis_index("s") * NC + lax.axis_index("c")
    b0 = wid * BW

    pltpu.sync_copy(idx_hbm.at[:, pl.ds(b0, BW)], idx_v)

    iota = lax.iota(jnp.int32, L)
    zero = jnp.zeros((L,), jnp.int32)
    _dt = lax.shift_right_logical(iota, 3)
    _r = jnp.bitwise_and(iota, 7)
    sb = [_dt * (8 * CP) + _r * CP + (2 * k) * (8 * CP) for k in range(D // L)]

    def prep_and_fire(s, buf):
        for c in range(BW // L):
            jidx_v[buf, pl.ds(c * L, L)] = idx_v[s, pl.ds(c * L, L)]
        pltpu.async_copy(tt_hbm.at[jidx_v.at[buf]], big_v.at[buf], gsem)
        pltpu.async_copy(pos_hbm.at[s], posrow_v.at[buf], psem)

    prep_and_fire(0, 0)

    def pair_body(s2, carry):
        for buf in range(2):
            s = s2 * 2 + buf
            osem = osem0 if buf == 0 else osem1

            @pl.when(s < S - 1)
            def _():
                prep_and_fire(s + 1, 1 - buf)

            pltpu.make_async_copy(
                tt_hbm.at[jidx_v.at[buf]], big_v.at[buf], gsem
            ).wait()
            pltpu.make_async_copy(pos_hbm.at[s], posrow_v.at[buf], psem).wait()

            @pl.when(s >= 2)
            def _():
                pltpu.make_async_copy(
                    oslab_v.at[buf, :, :, pl.ds(0, BW)],
                    out_hbm.at[s - 2, :, wid],
                    osem,
                ).wait()

            pos_k = [posrow_v[buf, pl.ds(k * L, L)] for k in range(D // L)]

            def c_body(c):
                base = lax.broadcast(c * L, (L,))
                idxk = [sb[k] + base for k in range(D // L)]
                for j in range(L):
                    t = c * L + j
                    for k in range(D // L):
                        v = big_v[buf, t, pl.ds(k * L, L)] + pos_k[k]
                        plsc.store_scatter(
                            oslab_v.at[buf], [zero, zero, idxk[k] + j], v
                        )

            plsc.parallel_loop(0, BW // L, 1, unroll=2)(c_body)

            pltpu.async_copy(
                oslab_v.at[buf, :, :, pl.ds(0, BW)], out_hbm.at[s, :, wid], osem
            )
        return carry

    lax.fori_loop(0, S // 2, pair_body, 0)

    pltpu.make_async_copy(
        oslab_v.at[0, :, :, pl.ds(0, BW)], out_hbm.at[S - 2, :, wid], osem0
    ).wait()
    pltpu.make_async_copy(
        oslab_v.at[1, :, :, pl.ds(0, BW)], out_hbm.at[S - 1, :, wid], osem1
    ).wait()


def kernel(inputs, token_table, pos_table):
    idx_t = inputs.T.astype(jnp.int32)
    tt = jnp.pad(token_table, ((0, 0), (0, D)))
    out5 = _emb_kernel(idx_t, tt, pos_table)
    return out5.transpose(2, 4, 0, 1, 3).reshape(B, S, D)


# speedup vs baseline: 1.7290x; 1.0896x over previous
import functools

import jax
import jax.numpy as jnp
from jax import lax
from jax.experimental import pallas as pl
from jax.experimental.pallas import tpu as pltpu
from jax.experimental.pallas import tpu_sc as plsc

B = 4096
S = 200
D = 64
V = 1000000

NC, NS = 2, 16
NW = NC * NS
BW = B // NW
L = 16
CP = BW + 1

_mesh = plsc.VectorSubcoreMesh(
    core_axis_name="c", subcore_axis_name="s", num_cores=NC, num_subcores=NS
)


@functools.partial(
    pl.kernel,
    out_type=jax.ShapeDtypeStruct((S, D // 8, NW, 8, BW), jnp.float32),
    mesh=_mesh,
    compiler_params=pltpu.CompilerParams(
        use_tc_tiling_on_sc=False, needs_layout_passes=False
    ),
    scratch_types=[
        pltpu.VMEM((S, BW), jnp.int32),
        pltpu.VMEM((2, BW), jnp.int32),
        pltpu.VMEM((2, BW, 2 * D), jnp.float32),
        pltpu.VMEM((2, D // 8, 8, CP), jnp.float32),
        pltpu.VMEM((2, D), jnp.float32),
        pltpu.SemaphoreType.DMA,
        pltpu.SemaphoreType.DMA,
        pltpu.SemaphoreType.DMA,
        pltpu.SemaphoreType.DMA,
    ],
)
def _emb_kernel(idx_hbm, tt_hbm, pos_hbm, out_hbm,
                idx_v, jidx_v, big_v, oslab_v, posrow_v,
                gsem, psem, osem0, osem1):
    wid = lax.axis_index("s") * NC + lax.axis_index("c")
    b0 = wid * BW

    pltpu.sync_copy(idx_hbm.at[:, pl.ds(b0, BW)], idx_v)

    iota = lax.iota(jnp.int32, L)
    rv = jnp.bitwise_and(iota, 7)
    dtv = [lax.shift_right_logical(iota, 3) + 2 * k for k in range(D // L)]

    def prep_and_fire(s, buf):
        for c in range(BW // L):
            jidx_v[buf, pl.ds(c * L, L)] = idx_v[s, pl.ds(c * L, L)]
        pltpu.async_copy(tt_hbm.at[jidx_v.at[buf]], big_v.at[buf], gsem)
        pltpu.async_copy(pos_hbm.at[s], posrow_v.at[buf], psem)

    prep_and_fire(0, 0)

    def pair_body(s2, carry):
        for buf in range(2):
            s = s2 * 2 + buf
            osem = osem0 if buf == 0 else osem1

            @pl.when(s < S - 1)
            def _():
                prep_and_fire(s + 1, 1 - buf)

            pltpu.make_async_copy(
                tt_hbm.at[jidx_v.at[buf]], big_v.at[buf], gsem
            ).wait()
            pltpu.make_async_copy(pos_hbm.at[s], posrow_v.at[buf], psem).wait()

            @pl.when(s >= 2)
            def _():
                pltpu.make_async_copy(
                    oslab_v.at[buf, :, :, pl.ds(0, BW)],
                    out_hbm.at[s - 2, :, wid],
                    osem,
                ).wait()

            pos_k = [posrow_v[buf, pl.ds(k * L, L)] for k in range(D // L)]

            def c_body(c):
                base = lax.broadcast(c * L, (L,))
                for j in range(L):
                    t = c * L + j
                    tv = base + j
                    for k in range(D // L):
                        v = big_v[buf, t, pl.ds(k * L, L)] + pos_k[k]
                        plsc.store_scatter(
                            oslab_v.at[buf], [dtv[k], rv, tv], v
                        )

            plsc.parallel_loop(0, BW // L, 1, unroll=2)(c_body)

            pltpu.async_copy(
                oslab_v.at[buf, :, :, pl.ds(0, BW)], out_hbm.at[s, :, wid], osem
            )
        return carry

    lax.fori_loop(0, S // 2, pair_body, 0)

    pltpu.make_async_copy(
        oslab_v.at[0, :, :, pl.ds(0, BW)], out_hbm.at[S - 2, :, wid], osem0
    ).wait()
    pltpu.make_async_copy(
        oslab_v.at[1, :, :, pl.ds(0, BW)], out_hbm.at[S - 1, :, wid], osem1
    ).wait()


def kernel(inputs, token_table, pos_table):
    idx_t = inputs.T.astype(jnp.int32)
    tt = jnp.pad(token_table, ((0, 0), (0, D)))
    out5 = _emb_kernel(idx_t, tt, pos_table)
    return out5.transpose(2, 4, 0, 1, 3).reshape(B, S, D)


# speedup vs baseline: 1.8001x; 1.0411x over previous
import functools

import jax
import jax.numpy as jnp
from jax import lax
from jax.experimental import pallas as pl
from jax.experimental.pallas import tpu as pltpu
from jax.experimental.pallas import tpu_sc as plsc

B = 4096
S = 200
D = 64
V = 1000000

NC, NS = 2, 16
NW = NC * NS
BW = B // NW
L = 16
CP = BW + 1

_mesh = plsc.VectorSubcoreMesh(
    core_axis_name="c", subcore_axis_name="s", num_cores=NC, num_subcores=NS
)


@functools.partial(
    pl.kernel,
    out_type=jax.ShapeDtypeStruct((S, D // 8, NW, 8, BW), jnp.float32),
    mesh=_mesh,
    compiler_params=pltpu.CompilerParams(
        use_tc_tiling_on_sc=False, needs_layout_passes=False
    ),
    scratch_types=[
        pltpu.VMEM((S, BW), jnp.int32),
        pltpu.VMEM((2, BW), jnp.int32),
        pltpu.VMEM((2, BW, D), jnp.float32),
        pltpu.VMEM((2, D // 8, 8, CP), jnp.float32),
        pltpu.VMEM((2, D), jnp.float32),
        pltpu.SemaphoreType.DMA,
        pltpu.SemaphoreType.DMA,
        pltpu.SemaphoreType.DMA,
        pltpu.SemaphoreType.DMA,
    ],
)
def _emb_kernel(idx_hbm, tt_hbm, pos_hbm, out_hbm,
                idx_v, jidx_v, big_v, oslab_v, posrow_v,
                gsem, psem, osem0, osem1):
    wid = lax.axis_index("s") * NC + lax.axis_index("c")
    b0 = wid * BW

    pltpu.sync_copy(idx_hbm.at[:, pl.ds(b0, BW)], idx_v)

    iota = lax.iota(jnp.int32, L)
    rv = jnp.bitwise_and(iota, 7)
    dtv = [lax.shift_right_logical(iota, 3) + 2 * k for k in range(D // L)]

    def prep_and_fire(s, buf):
        for c in range(BW // L):
            jidx_v[buf, pl.ds(c * L, L)] = lax.shift_left(
                idx_v[s, pl.ds(c * L, L)], 1
            )
        pltpu.async_copy(tt_hbm.at[jidx_v.at[buf]], big_v.at[buf], gsem)
        pltpu.async_copy(pos_hbm.at[s], posrow_v.at[buf], psem)

    prep_and_fire(0, 0)

    def pair_body(s2, carry):
        for buf in range(2):
            s = s2 * 2 + buf
            osem = osem0 if buf == 0 else osem1

            @pl.when(s < S - 1)
            def _():
                prep_and_fire(s + 1, 1 - buf)

            pltpu.make_async_copy(
                tt_hbm.at[jidx_v.at[buf]], big_v.at[buf], gsem
            ).wait()
            pltpu.make_async_copy(pos_hbm.at[s], posrow_v.at[buf], psem).wait()

            @pl.when(s >= 2)
            def _():
                pltpu.make_async_copy(
                    oslab_v.at[buf, :, :, pl.ds(0, BW)],
                    out_hbm.at[s - 2, :, wid],
                    osem,
                ).wait()

            pos_k = [posrow_v[buf, pl.ds(k * L, L)] for k in range(D // L)]

            def c_body(c):
                for j in range(L):
                    t = c * L + j
                    tv = lax.broadcast(t, (L,))
                    for k in range(D // L):
                        v = big_v[buf, t, pl.ds(k * L, L)] + pos_k[k]
                        plsc.store_scatter(
                            oslab_v.at[buf], [dtv[k], rv, tv], v
                        )

            plsc.parallel_loop(0, BW // L, 1, unroll=2)(c_body)

            pltpu.async_copy(
                oslab_v.at[buf, :, :, pl.ds(0, BW)], out_hbm.at[s, :, wid], osem
            )
        return carry

    lax.fori_loop(0, S // 2, pair_body, 0)

    pltpu.make_async_copy(
        oslab_v.at[0, :, :, pl.ds(0, BW)], out_hbm.at[S - 2, :, wid], osem0
    ).wait()
    pltpu.make_async_copy(
        oslab_v.at[1, :, :, pl.ds(0, BW)], out_hbm.at[S - 1, :, wid], osem1
    ).wait()


def kernel(inputs, token_table, pos_table):
    idx_t = inputs.T.astype(jnp.int32)
    tt = jnp.pad(token_table, ((0, 0), (0, D))).reshape(2 * V, D)
    out5 = _emb_kernel(idx_t, tt, pos_table)
    return out5.transpose(2, 4, 0, 1, 3).reshape(B, S, D)


# speedup vs baseline: 2.2229x; 1.2349x over previous
import functools

import jax
import jax.numpy as jnp
from jax import lax
from jax.experimental import pallas as pl
from jax.experimental.pallas import tpu as pltpu
from jax.experimental.pallas import tpu_sc as plsc

B = 4096
S = 200
D = 64
V = 1000000

NC, NS = 2, 16
NW = NC * NS
BW = B // NW
L = 16
CP = BW + 1

_mesh = plsc.VectorSubcoreMesh(
    core_axis_name="c", subcore_axis_name="s", num_cores=NC, num_subcores=NS
)


@functools.partial(
    pl.kernel,
    out_type=jax.ShapeDtypeStruct((S, D // 8, NW, 8, BW), jnp.float32),
    mesh=_mesh,
    compiler_params=pltpu.CompilerParams(
        use_tc_tiling_on_sc=False, needs_layout_passes=False
    ),
    scratch_types=[
        pltpu.VMEM((S, BW), jnp.int32),
        pltpu.VMEM((2, BW), jnp.int32),
        pltpu.VMEM((2, BW, D), jnp.float32),
        pltpu.VMEM((2, D // 8, 8, CP), jnp.float32),
        pltpu.VMEM((2, D), jnp.float32),
        pltpu.SemaphoreType.DMA,
        pltpu.SemaphoreType.DMA,
        pltpu.SemaphoreType.DMA,
        pltpu.SemaphoreType.DMA,
    ],
)
def _emb_kernel(idx_hbm, tt_hbm, pos_hbm, out_hbm,
                idx_v, jidx_v, big_v, oslab_v, posrow_v,
                gsem, psem, osem0, osem1):
    wid = lax.axis_index("s") * NC + lax.axis_index("c")
    b0 = wid * BW

    pltpu.sync_copy(idx_hbm.at[:, pl.ds(b0, BW)], idx_v)

    iota = lax.iota(jnp.int32, L)
    rv = jnp.bitwise_and(iota, 7)
    dtv = [lax.shift_right_logical(iota, 3) + 2 * k for k in range(D // L)]

    def prep_and_fire(s, buf):
        for c in range(BW // L):
            jidx_v[buf, pl.ds(c * L, L)] = lax.shift_left(
                idx_v[s, pl.ds(c * L, L)], 1
            )
        pltpu.async_copy(tt_hbm.at[jidx_v.at[buf]], big_v.at[buf], gsem)
        pltpu.async_copy(pos_hbm.at[s], posrow_v.at[buf], psem)

    prep_and_fire(0, 0)

    def pair_body(s2, carry):
        for buf in range(2):
            s = s2 * 2 + buf
            osem = osem0 if buf == 0 else osem1

            @pl.when(s < S - 1)
            def _():
                prep_and_fire(s + 1, 1 - buf)

            pltpu.make_async_copy(
                tt_hbm.at[jidx_v.at[buf]], big_v.at[buf], gsem
            ).wait()
            pltpu.make_async_copy(pos_hbm.at[s], posrow_v.at[buf], psem).wait()

            @pl.when(s >= 2)
            def _():
                pltpu.make_async_copy(
                    oslab_v.at[buf, :, :, pl.ds(0, BW)],
                    out_hbm.at[s - 2, :, wid],
                    osem,
                ).wait()

            pos_k = [posrow_v[buf, pl.ds(k * L, L)] for k in range(D // L)]

            def c_body(c):
                for j in range(L):
                    t = c * L + j
                    tv = lax.broadcast(t, (L,))
                    for k in range(D // L):
                        v = big_v[buf, t, pl.ds(k * L, L)] + pos_k[k]
                        plsc.store_scatter(
                            oslab_v.at[buf], [dtv[k], rv, tv], v
                        )

            plsc.parallel_loop(0, BW // L, 1, unroll=4)(c_body)

            pltpu.async_copy(
                oslab_v.at[buf, :, :, pl.ds(0, BW)], out_hbm.at[s, :, wid], osem
            )
        return carry

    lax.fori_loop(0, S // 2, pair_body, 0)

    pltpu.make_async_copy(
        oslab_v.at[0, :, :, pl.ds(0, BW)], out_hbm.at[S - 2, :, wid], osem0
    ).wait()
    pltpu.make_async_copy(
        oslab_v.at[1, :, :, pl.ds(0, BW)], out_hbm.at[S - 1, :, wid], osem1
    ).wait()


def kernel(inputs, token_table, pos_table):
    idx_t = inputs.T.astype(jnp.int32)
    tt = jnp.pad(token_table, ((0, 0), (0, D))).reshape(2 * V, D)
    out5 = _emb_kernel(idx_t, tt, pos_table)
    return out5.transpose(2, 4, 0, 1, 3).reshape(B, S, D)
